# baseline (speedup 1.0000x reference)
import functools

import jax
import jax.numpy as jnp
from jax import lax
from jax.experimental import pallas as pl
from jax.experimental.pallas import tpu as pltpu
from jax.experimental.pallas import tpu_sc as plsc

N = 10000
H = 128
E = 320000
NW = 32
LANES = 16
TPW = 79
EPW = TPW * 128
NP = 10240
SEG = 1024
SENT = N + 16


def _tc_scores_body(w_ref, h_ref, b_ref, o_ref):
    acc = jax.lax.dot_general(
        w_ref[...], h_ref[...], (((1,), (1,)), ((), ())),
        preferred_element_type=jnp.float32) + b_ref[:, :1]
    o_ref[0, :, pl.ds(0, N)] = acc[0:4, :]


def _tc_softmax_body(p_ref, sb_ref, o_ref):
    x = p_ref[0] + p_ref[1] + sb_ref[...]
    ridx = lax.broadcasted_iota(jnp.int32, x.shape, 0)
    lidx = lax.broadcasted_iota(jnp.int32, x.shape, 1)
    valid = ridx * 128 + lidx < N
    x = jnp.where(valid, x, -jnp.inf)
    m = jnp.max(x)
    e = jnp.exp(x - m)
    s = jnp.sum(e)
    o_ref[...] = e * (1.0 / s)


def _sc_edge_body(sc_hbm, ei_hbm, z_hbm, out_hbm, sb_hbm,
                  sc_v, ei_v, acc_l, red_v, res_v, sb_v, acc_sh, sem):
    cid = lax.axis_index("c")
    sid = lax.axis_index("s")
    wid = sid * 2 + cid

    t0 = pl.multiple_of((78 * wid + jnp.minimum(wid, 4)) * 128, 128)

    cps = pltpu.async_copy(sc_hbm.at[0], sc_v, sem)
    cpz = pltpu.async_copy(z_hbm, acc_l, sem)

    @pl.when(wid < 4)
    def _stage_full():
        pltpu.sync_copy(ei_hbm.at[:, pl.ds(t0, EPW)], ei_v)

    @pl.when(wid >= 4)
    def _stage_part():
        pltpu.sync_copy(ei_hbm.at[:, pl.ds(t0, EPW - 128)],
                        ei_v.at[:, pl.ds(0, EPW - 128)])
        sent = jnp.full((LANES,), SENT, jnp.int32)
        for l in range(128 // LANES):
            ei_v[0, pl.ds(EPW - 128 + l * LANES, LANES)] = sent
            ei_v[1, pl.ds(EPW - 128 + l * LANES, LANES)] = sent

    cps.wait()
    cpz.wait()

    row0 = jnp.zeros((LANES,), jnp.int32)
    row1 = jnp.full((LANES,), 1, jnp.int32)

    @plsc.parallel_loop(0, EPW, LANES, unroll=16)
    def _grp(o):
        si = ei_v[0, pl.ds(o, LANES)]
        di = ei_v[1, pl.ds(o, LANES)]
        va = plsc.load_gather(sc_v, [row0, si])
        vc = plsc.load_gather(sc_v, [row1, di])
        plsc.addupdate_scatter(acc_l, [di], va + vc)

    pltpu.sync_copy(acc_l, acc_sh.at[sid])
    plsc.subcore_barrier()

    @pl.when(sid < NP // SEG)
    def _reduce():
        pltpu.sync_copy(acc_sh.at[:, pl.ds(sid * SEG, SEG)], red_v)
        for g in range(SEG // LANES):
            tot = red_v[0, pl.ds(g * LANES, LANES)]
            for r in range(1, 16):
                tot = tot + red_v[r, pl.ds(g * LANES, LANES)]
            res_v[g // 8, pl.ds((g % 8) * LANES, LANES)] = tot

        pltpu.sync_copy(res_v, out_hbm.at[cid, pl.ds(sid * 8, 8), :])

    @pl.when((cid == 0) & (sid < NP // SEG))
    def _sb():
        for g in range(SEG // LANES):
            sb_v[g // 8, pl.ds((g % 8) * LANES, LANES)] = (
                sc_v[2, pl.ds(sid * SEG + g * LANES, LANES)])
        pltpu.sync_copy(sb_v, sb_hbm.at[pl.ds(sid * 8, 8), :])


@functools.cache
def _sc_edge():
    return pl.kernel(
        _sc_edge_body,
        out_type=(jax.ShapeDtypeStruct((2, NP // 128, 128), jnp.float32),
                  jax.ShapeDtypeStruct((NP // 128, 128), jnp.float32)),
        mesh=plsc.VectorSubcoreMesh(core_axis_name="c", subcore_axis_name="s"),
        compiler_params=pltpu.CompilerParams(needs_layout_passes=False),
        scratch_types=[
            pltpu.VMEM((4, NP), jnp.float32),
            pltpu.VMEM((2, EPW), jnp.int32),
            pltpu.VMEM((NP,), jnp.float32),
            pltpu.VMEM((16, SEG), jnp.float32),
            pltpu.VMEM((8, 128), jnp.float32),
            pltpu.VMEM((8, 128), jnp.float32),
            pltpu.MemorySpace.VMEM_SHARED((16, NP), jnp.float32),
            pltpu.SemaphoreType.DMA,
        ],
    )


@jax.jit
def kernel(h, edge_index, W_edge, b_edge, W_node, b_node):
    h = h.astype(jnp.float32)
    ei = edge_index.astype(jnp.int32)

    w3 = jnp.zeros((8, H), jnp.float32)
    w3 = w3.at[0].set(W_edge[:H]).at[1].set(W_edge[H:]).at[2].set(W_node)
    bias = jnp.zeros((8, 1), jnp.float32)
    bias = bias.at[1, 0].set(b_edge).at[2, 0].set(b_node)
    bias = jnp.broadcast_to(bias, (8, 128))

    scores = pl.pallas_call(
        _tc_scores_body,
        out_shape=jax.ShapeDtypeStruct((2, 4, NP), jnp.float32),
    )(w3, h, bias)

    zeros = jnp.zeros((NP,), jnp.float32)
    parts, sb = _sc_edge()(scores, ei, zeros)

    out = pl.pallas_call(
        _tc_softmax_body,
        out_shape=jax.ShapeDtypeStruct((NP // 128, 128), jnp.float32),
    )(parts, sb)

    return out.reshape(NP)[:N]


# speedup vs baseline: 1.0739x; 1.0739x over previous
import functools

import jax
import jax.numpy as jnp
from jax import lax
from jax.experimental import pallas as pl
from jax.experimental.pallas import tpu as pltpu
from jax.experimental.pallas import tpu_sc as plsc

N = 10000
H = 128
E = 320000
NW = 32
LANES = 16
TPW = 79
EPW = TPW * 128
NP = 10240
SEG = 1024
SENT = N + 16


def _tc_scores_body(w_ref, h_ref, b_ref, o_ref):
    acc = jax.lax.dot_general(
        w_ref[...], h_ref[...], (((1,), (1,)), ((), ())),
        preferred_element_type=jnp.float32) + b_ref[:, :1]
    o_ref[0, :, pl.ds(0, N)] = acc[0:4, :]


def _tc_softmax_body(p_ref, sb_ref, o_ref):
    x = p_ref[0] + p_ref[1] + sb_ref[...]
    ridx = lax.broadcasted_iota(jnp.int32, x.shape, 0)
    lidx = lax.broadcasted_iota(jnp.int32, x.shape, 1)
    valid = ridx * 128 + lidx < N
    x = jnp.where(valid, x, -jnp.inf)
    m = jnp.max(x)
    e = jnp.exp(x - m)
    s = jnp.sum(e)
    o_ref[...] = e * (1.0 / s)


def _sc_edge_body(sc_hbm, ei_hbm, out_hbm, sb_hbm,
                  sc_v, ei_v, acc_l, red_v, res_v, sb_v, acc_sh, sem):
    cid = lax.axis_index("c")
    sid = lax.axis_index("s")
    wid = sid * 2 + cid

    t0 = pl.multiple_of((78 * wid + jnp.minimum(wid, 4)) * 128, 128)

    cps = pltpu.async_copy(sc_hbm.at[0], sc_v, sem)

    @pl.when(wid < 4)
    def _stage_full():
        pltpu.sync_copy(ei_hbm.at[:, pl.ds(t0, EPW)], ei_v)

    @pl.when(wid >= 4)
    def _stage_part():
        pltpu.sync_copy(ei_hbm.at[:, pl.ds(t0, EPW - 128)],
                        ei_v.at[:, pl.ds(0, EPW - 128)])
        sent = jnp.full((LANES,), SENT, jnp.int32)
        for l in range(128 // LANES):
            ei_v[0, pl.ds(EPW - 128 + l * LANES, LANES)] = sent
            ei_v[1, pl.ds(EPW - 128 + l * LANES, LANES)] = sent

    fz = jnp.zeros((LANES,), jnp.float32)

    @plsc.parallel_loop(0, NP, LANES, unroll=8)
    def _zero(o):
        acc_l[pl.ds(o, LANES)] = fz

    cps.wait()

    row0 = jnp.zeros((LANES,), jnp.int32)
    row1 = jnp.full((LANES,), 1, jnp.int32)

    @plsc.parallel_loop(0, EPW, LANES, unroll=8)
    def _grp(o):
        si = ei_v[0, pl.ds(o, LANES)]
        di = ei_v[1, pl.ds(o, LANES)]
        va = plsc.load_gather(sc_v, [row0, si])
        vc = plsc.load_gather(sc_v, [row1, di])
        plsc.addupdate_scatter(acc_l, [di], va + vc)

    pltpu.sync_copy(acc_l, acc_sh.at[sid])
    plsc.subcore_barrier()

    @pl.when(sid < NP // SEG)
    def _reduce():
        pltpu.sync_copy(acc_sh.at[:, pl.ds(sid * SEG, SEG)], red_v)
        for g in range(SEG // LANES):
            tot = red_v[0, pl.ds(g * LANES, LANES)]
            for r in range(1, 16):
                tot = tot + red_v[r, pl.ds(g * LANES, LANES)]
            res_v[g // 8, pl.ds((g % 8) * LANES, LANES)] = tot

        pltpu.sync_copy(res_v, out_hbm.at[cid, pl.ds(sid * 8, 8), :])

    @pl.when((cid == 0) & (sid < NP // SEG))
    def _sb():
        for g in range(SEG // LANES):
            sb_v[g // 8, pl.ds((g % 8) * LANES, LANES)] = (
                sc_v[2, pl.ds(sid * SEG + g * LANES, LANES)])
        pltpu.sync_copy(sb_v, sb_hbm.at[pl.ds(sid * 8, 8), :])


@functools.cache
def _sc_edge():
    return pl.kernel(
        _sc_edge_body,
        out_type=(jax.ShapeDtypeStruct((2, NP // 128, 128), jnp.float32),
                  jax.ShapeDtypeStruct((NP // 128, 128), jnp.float32)),
        mesh=plsc.VectorSubcoreMesh(core_axis_name="c", subcore_axis_name="s"),
        compiler_params=pltpu.CompilerParams(needs_layout_passes=False),
        scratch_types=[
            pltpu.VMEM((4, NP), jnp.float32),
            pltpu.VMEM((2, EPW), jnp.int32),
            pltpu.VMEM((NP,), jnp.float32),
            pltpu.VMEM((16, SEG), jnp.float32),
            pltpu.VMEM((8, 128), jnp.float32),
            pltpu.VMEM((8, 128), jnp.float32),
            pltpu.MemorySpace.VMEM_SHARED((16, NP), jnp.float32),
            pltpu.SemaphoreType.DMA,
        ],
    )


@jax.jit
def kernel(h, edge_index, W_edge, b_edge, W_node, b_node):
    h = h.astype(jnp.float32)
    ei = edge_index.astype(jnp.int32)

    w3 = jnp.zeros((8, H), jnp.float32)
    w3 = w3.at[0].set(W_edge[:H]).at[1].set(W_edge[H:]).at[2].set(W_node)
    bias = jnp.zeros((8, 1), jnp.float32)
    bias = bias.at[1, 0].set(b_edge).at[2, 0].set(b_node)
    bias = jnp.broadcast_to(bias, (8, 128))

    scores = pl.pallas_call(
        _tc_scores_body,
        out_shape=jax.ShapeDtypeStruct((2, 4, NP), jnp.float32),
    )(w3, h, bias)

    parts, sb = _sc_edge()(scores, ei)

    out = pl.pallas_call(
        _tc_softmax_body,
        out_shape=jax.ShapeDtypeStruct((NP // 128, 128), jnp.float32),
    )(parts, sb)

    return out.reshape(NP)[:N]
